# baseline (speedup 1.0000x reference)
import functools

import jax
import jax.numpy as jnp
import numpy as np
from jax import lax
from jax.experimental import pallas as pl

_B, _N, _G = 8, 20000, 64
_POS_THR, _NEG_THR = 0.5, 0.1
_SAMPLES = 512
_POS_QUOTA = 128
_L1W = 1.0

_BLK = 2048
_NP = 20480


def _iou_body(pr_ref, gt_ref, mx_ref, am_ref):
    pr = pr_ref[0]
    gt = gt_ref[0]
    px1 = pr[0:1, :]
    py1 = pr[1:2, :]
    px2 = pr[2:3, :]
    py2 = pr[3:4, :]
    gx1 = gt[:, 0:1]
    gy1 = gt[:, 1:2]
    gx2 = gt[:, 2:3]
    gy2 = gt[:, 3:4]
    x1 = jnp.maximum(px1, gx1)
    y1 = jnp.maximum(py1, gy1)
    x2 = jnp.minimum(px2, gx2)
    y2 = jnp.minimum(py2, gy2)
    inter = jnp.maximum(x2 - x1, 0.0) * jnp.maximum(y2 - y1, 0.0)
    ap = (px2 - px1) * (py2 - py1)
    ag = (gx2 - gx1) * (gy2 - gy1)
    union = ap + ag - inter
    iou = inter / jnp.maximum(union, 1e-8)
    mx = jnp.max(iou, axis=0, keepdims=True)
    gidx = lax.broadcasted_iota(jnp.int32, (_G, 1), 0)
    am = jnp.min(jnp.where(iou == mx, gidx, _G), axis=0, keepdims=True)
    mx_ref[...] = mx[None]
    am_ref[...] = am[None]


def _max_argmax(proposals, gt_boxes):
    pr_t = jnp.transpose(proposals, (0, 2, 1))
    pr_t = jnp.pad(pr_t, ((0, 0), (0, 0), (0, _NP - _N)))
    grid = (_B, _NP // _BLK)
    mx, am = pl.pallas_call(
        _iou_body,
        grid=grid,
        in_specs=[
            pl.BlockSpec((1, 4, _BLK), lambda b, n: (b, 0, n)),
            pl.BlockSpec((1, _G, 4), lambda b, n: (b, 0, 0)),
        ],
        out_specs=[
            pl.BlockSpec((1, 1, _BLK), lambda b, n: (b * (_NP // _BLK) + n, 0, 0)),
            pl.BlockSpec((1, 1, _BLK), lambda b, n: (b * (_NP // _BLK) + n, 0, 0)),
        ],
        out_shape=[
            jax.ShapeDtypeStruct((_B * (_NP // _BLK), 1, _BLK), jnp.float32),
            jax.ShapeDtypeStruct((_B * (_NP // _BLK), 1, _BLK), jnp.int32),
        ],
    )(pr_t, gt_boxes)
    mx = mx.reshape(_B, _NP)
    am = am.reshape(_B, _NP)
    return mx[:, :_N], am[:, :_N]


def _centrehw(b):
    w = b[..., 2] - b[..., 0]
    h = b[..., 3] - b[..., 1]
    return jnp.stack([b[..., 0] + 0.5 * w, b[..., 1] + 0.5 * h, w, h], axis=-1)


def kernel(proposals, gt_boxes):
    max_iou, argmax_gt = _max_argmax(proposals, gt_boxes)

    idx_dtype = jnp.int32
    labels = jnp.where(max_iou >= _POS_THR, 1,
                       jnp.where(max_iou < _NEG_THR, 0, -1))
    pos_mask = labels == 1
    neg_mask = labels == 0
    pos_key = jnp.where(pos_mask, -max_iou, jnp.inf)
    neg_key = jnp.where(neg_mask, max_iou, jnp.inf)
    pos_order = jnp.argsort(pos_key, axis=1, stable=True)
    neg_order = jnp.argsort(neg_key, axis=1, stable=True)
    pos_rank = jnp.argsort(pos_order, axis=1)
    neg_rank = jnp.argsort(neg_order, axis=1)
    n_pos = jnp.minimum(jnp.sum(pos_mask, axis=1), _POS_QUOTA)[:, None]
    n_neg = jnp.minimum(jnp.sum(neg_mask, axis=1)[:, None], _SAMPLES - n_pos)
    sel_pos = pos_mask & (pos_rank < n_pos)
    sel_neg = neg_mask & (neg_rank < n_neg)
    unsel = ~(sel_pos | sel_neg)
    fill_rank = jnp.cumsum(unsel.astype(idx_dtype), axis=1) - 1
    n_fill = _SAMPLES - n_pos - n_neg
    sel_fill = unsel & (fill_rank < n_fill)
    slot = jnp.where(sel_pos, pos_rank,
           jnp.where(sel_neg, n_pos + neg_rank,
           jnp.where(sel_fill, n_pos + n_neg + fill_rank, _SAMPLES)))
    bsz, n = proposals.shape[0], proposals.shape[1]
    rows = jnp.broadcast_to(jnp.arange(bsz)[:, None], (bsz, n))
    cols = jnp.broadcast_to(jnp.arange(n, dtype=idx_dtype)[None, :], (bsz, n))
    keep_idx = jnp.zeros((bsz, _SAMPLES), idx_dtype).at[rows, slot].set(
        cols, mode='drop')
    batch_labels = (jnp.arange(_SAMPLES)[None, :] < n_pos).astype(idx_dtype)
    b_idx = jnp.arange(bsz)[:, None]
    ag = argmax_gt[b_idx, keep_idx]

    roi_batch = proposals[b_idx, keep_idx]
    gt_batch = gt_boxes[b_idx, ag]
    gt_c = _centrehw(gt_batch)
    roi_c = _centrehw(roi_batch)
    dx = (gt_c[..., 0] - roi_c[..., 0]) / roi_c[..., 2]
    dy = (gt_c[..., 1] - roi_c[..., 1]) / roi_c[..., 3]
    dw = jnp.log(gt_c[..., 2] / roi_c[..., 2])
    dh = jnp.log(gt_c[..., 3] / roi_c[..., 3])
    bbox_targets = jnp.stack([dx, dy, dw, dh], axis=-1)
    in_weights = jnp.where((batch_labels == 1)[..., None], _L1W, 0.0) * \
        jnp.ones((1, 1, 4), jnp.float32)
    out_weights = (in_weights > 0).astype(jnp.float32)
    return (roi_batch, batch_labels, bbox_targets, in_weights, out_weights)


# speedup vs baseline: 1.2360x; 1.2360x over previous
import functools

import jax
import jax.numpy as jnp
import numpy as np
from jax import lax
from jax.experimental import pallas as pl

_B, _N, _G = 8, 20000, 64
_POS_THR, _NEG_THR = 0.5, 0.1
_SAMPLES = 512
_POS_QUOTA = 128
_L1W = 1.0

_BLK = 2048
_NP = 20480


def _iou_body(pr_ref, gt_ref, mx_ref, am_ref):
    pr = pr_ref[0]
    gt = gt_ref[0]
    px1 = pr[0:1, :]
    py1 = pr[1:2, :]
    px2 = pr[2:3, :]
    py2 = pr[3:4, :]
    gx1 = gt[:, 0:1]
    gy1 = gt[:, 1:2]
    gx2 = gt[:, 2:3]
    gy2 = gt[:, 3:4]
    x1 = jnp.maximum(px1, gx1)
    y1 = jnp.maximum(py1, gy1)
    x2 = jnp.minimum(px2, gx2)
    y2 = jnp.minimum(py2, gy2)
    inter = jnp.maximum(x2 - x1, 0.0) * jnp.maximum(y2 - y1, 0.0)
    ap = (px2 - px1) * (py2 - py1)
    ag = (gx2 - gx1) * (gy2 - gy1)
    union = ap + ag - inter
    iou = inter / jnp.maximum(union, 1e-8)
    mx = jnp.max(iou, axis=0, keepdims=True)
    gidx = lax.broadcasted_iota(jnp.int32, (_G, 1), 0)
    am = jnp.min(jnp.where(iou == mx, gidx, _G), axis=0, keepdims=True)
    mx_ref[...] = mx[None]
    am_ref[...] = am[None]


def _max_argmax(proposals, gt_boxes):
    pr_t = jnp.transpose(proposals, (0, 2, 1))
    pr_t = jnp.pad(pr_t, ((0, 0), (0, 0), (0, _NP - _N)))
    grid = (_B, _NP // _BLK)
    mx, am = pl.pallas_call(
        _iou_body,
        grid=grid,
        in_specs=[
            pl.BlockSpec((1, 4, _BLK), lambda b, n: (b, 0, n)),
            pl.BlockSpec((1, _G, 4), lambda b, n: (b, 0, 0)),
        ],
        out_specs=[
            pl.BlockSpec((1, 1, _BLK), lambda b, n: (b * (_NP // _BLK) + n, 0, 0)),
            pl.BlockSpec((1, 1, _BLK), lambda b, n: (b * (_NP // _BLK) + n, 0, 0)),
        ],
        out_shape=[
            jax.ShapeDtypeStruct((_B * (_NP // _BLK), 1, _BLK), jnp.float32),
            jax.ShapeDtypeStruct((_B * (_NP // _BLK), 1, _BLK), jnp.int32),
        ],
    )(pr_t, gt_boxes)
    mx = mx.reshape(_B, _NP)
    am = am.reshape(_B, _NP)
    return mx[:, :_N], am[:, :_N]


def _centrehw(b):
    w = b[..., 2] - b[..., 0]
    h = b[..., 3] - b[..., 1]
    return jnp.stack([b[..., 0] + 0.5 * w, b[..., 1] + 0.5 * h, w, h], axis=-1)


def kernel(proposals, gt_boxes):
    max_iou, argmax_gt = _max_argmax(proposals, gt_boxes)

    idx_dtype = jnp.int32
    pos_mask = max_iou >= _POS_THR
    neg_mask = max_iou < _NEG_THR
    n_pos = jnp.minimum(jnp.sum(pos_mask, axis=1), _POS_QUOTA)[:, None]
    n_neg = jnp.minimum(jnp.sum(neg_mask, axis=1)[:, None], _SAMPLES - n_pos)

    _, pos_idx = lax.top_k(jnp.where(pos_mask, max_iou, -1.0), _POS_QUOTA)
    _, neg_idx = lax.top_k(jnp.where(neg_mask, -max_iou, -2.0), _SAMPLES)

    _FILLW = 1280
    bsz = proposals.shape[0]
    b_idx = jnp.arange(bsz)[:, None]
    s_idx = jnp.arange(_SAMPLES)[None, :]
    unsel = jnp.ones((bsz, _FILLW), jnp.bool_)
    pos_in = (s_idx[:, :_POS_QUOTA] < n_pos) & (pos_idx < _FILLW)
    unsel = unsel.at[b_idx, jnp.where(pos_in, pos_idx, _FILLW)].set(
        False, mode='drop')
    neg_in = (s_idx < n_neg) & (neg_idx < _FILLW)
    unsel = unsel.at[b_idx, jnp.where(neg_in, neg_idx, _FILLW)].set(
        False, mode='drop')
    fill_key = jnp.where(unsel, -jnp.arange(_FILLW, dtype=idx_dtype)[None, :],
                         -_FILLW - 1)
    _, fill_idx = lax.top_k(fill_key, _SAMPLES)

    take = functools.partial(jnp.take_along_axis, axis=1)
    sn = jnp.clip(s_idx - n_pos, 0, _SAMPLES - 1)
    sf = jnp.clip(s_idx - n_pos - n_neg, 0, _SAMPLES - 1)
    keep_idx = jnp.where(
        s_idx < n_pos, take(pos_idx, jnp.clip(s_idx, 0, _POS_QUOTA - 1)),
        jnp.where(s_idx < n_pos + n_neg, take(neg_idx, sn),
                  take(fill_idx, sf))).astype(idx_dtype)
    batch_labels = (s_idx < n_pos).astype(idx_dtype)
    ag = argmax_gt[b_idx, keep_idx]

    roi_batch = proposals[b_idx, keep_idx]
    gt_batch = gt_boxes[b_idx, ag]
    gt_c = _centrehw(gt_batch)
    roi_c = _centrehw(roi_batch)
    dx = (gt_c[..., 0] - roi_c[..., 0]) / roi_c[..., 2]
    dy = (gt_c[..., 1] - roi_c[..., 1]) / roi_c[..., 3]
    dw = jnp.log(gt_c[..., 2] / roi_c[..., 2])
    dh = jnp.log(gt_c[..., 3] / roi_c[..., 3])
    bbox_targets = jnp.stack([dx, dy, dw, dh], axis=-1)
    in_weights = jnp.where((batch_labels == 1)[..., None], _L1W, 0.0) * \
        jnp.ones((1, 1, 4), jnp.float32)
    out_weights = (in_weights > 0).astype(jnp.float32)
    return (roi_batch, batch_labels, bbox_targets, in_weights, out_weights)
